# baseline (speedup 1.0000x reference)
import functools

import jax
import jax.numpy as jnp
from jax import lax
from jax.experimental import pallas as pl
from jax.experimental.pallas import tpu as pltpu
from jax.experimental.pallas import tpu_sc as plsc

NC = 2
NS = 16
L = 16
NW = NC * NS

B = 16384
D = 32
GROUP = 128
CHUNK = 128
BPW = B // NW
NCHUNK = BPW // CHUNK

_mesh = plsc.VectorSubcoreMesh(core_axis_name="c", subcore_axis_name="s")


@functools.partial(
    pl.kernel,
    mesh=_mesh,
    compiler_params=pltpu.CompilerParams(
        needs_layout_passes=False, use_tc_tiling_on_sc=True),
    out_type=jax.ShapeDtypeStruct((B,), jnp.float32),
    scratch_types=[
        pltpu.VMEM((BPW,), jnp.int32),
        pltpu.VMEM((BPW,), jnp.int32),
        pltpu.VMEM((NCHUNK, CHUNK), jnp.int32),
        pltpu.VMEM((NCHUNK, CHUNK), jnp.int32),
        pltpu.VMEM((BPW,), jnp.int32),
        pltpu.VMEM((BPW,), jnp.int32),
        pltpu.VMEM((2, CHUNK, GROUP), jnp.float32),
        pltpu.VMEM((2, CHUNK, GROUP), jnp.float32),
        pltpu.VMEM((BPW,), jnp.float32),
        pltpu.SemaphoreType.DMA,
        pltpu.SemaphoreType.DMA,
    ],
)
def _sc_embed_dot(x0_hbm, x1_hbm, tab_hbm, out_hbm,
                  x0_v, x1_v, idx0_v, idx1_v, off0_v, off1_v,
                  rows0_v, rows1_v, out_v, sem_a, sem_b):
    wid = lax.axis_index("s") * NC + lax.axis_index("c")
    base = wid * BPW

    pltpu.sync_copy(x0_hbm.at[pl.ds(base, BPW)], x0_v)
    pltpu.sync_copy(x1_hbm.at[pl.ds(base, BPW)], x1_v)

    for g in range(BPW // L):
        a = x0_v[pl.ds(g * L, L)]
        b = a + x1_v[pl.ds(g * L, L)]
        c = g // (CHUNK // L)
        j = g % (CHUNK // L)
        idx0_v[c, pl.ds(j * L, L)] = a >> 2
        idx1_v[c, pl.ds(j * L, L)] = b >> 2
        off0_v[pl.ds(g * L, L)] = (a & 3) * D
        off1_v[pl.ds(g * L, L)] = (b & 3) * D

    sems = (sem_a, sem_b)
    lanes = lax.iota(jnp.int32, L)

    def fire(c):
        slot = c % 2
        s = sems[slot]
        return (
            pltpu.async_copy(tab_hbm.at[idx0_v.at[c]], rows0_v.at[slot], s),
            pltpu.async_copy(tab_hbm.at[idx1_v.at[c]], rows1_v.at[slot], s),
        )

    pending = {0: fire(0)}
    for c in range(NCHUNK):
        if c + 1 < NCHUNK:
            pending[c + 1] = fire(c + 1)
        for h in pending.pop(c):
            h.wait()
        slot = c % 2

        def group_body(g, _, c=c, slot=slot):
            o0 = off0_v[pl.ds(c * CHUNK + g * L, L)]
            o1 = off1_v[pl.ds(c * CHUNK + g * L, L)]
            acc = jnp.zeros((L,), jnp.float32)
            for r in range(L):
                row = g * L + r
                s0 = o0[r]
                s1 = o1[r]
                a0 = rows0_v[slot, row, pl.ds(s0, L)]
                a1 = rows0_v[slot, row, pl.ds(s0 + L, L)]
                b0 = rows1_v[slot, row, pl.ds(s1, L)]
                b1 = rows1_v[slot, row, pl.ds(s1 + L, L)]
                s = jnp.sum(a0 * b0 + a1 * b1)
                acc = jnp.where(lanes == r, s, acc)
            out_v[pl.ds(c * CHUNK + g * L, L)] = acc
            return 0
        lax.fori_loop(0, CHUNK // L, group_body, 0)

    pltpu.sync_copy(out_v, out_hbm.at[pl.ds(base, BPW)])


def kernel(x, table):
    x0 = x[:, 0]
    x1 = x[:, 1]
    tab = table.reshape(250000, GROUP)
    return _sc_embed_dot(x0, x1, tab)


# speedup vs baseline: 1.0143x; 1.0143x over previous
import functools

import jax
import jax.numpy as jnp
from jax import lax
from jax.experimental import pallas as pl
from jax.experimental.pallas import tpu as pltpu
from jax.experimental.pallas import tpu_sc as plsc

NC = 2
NS = 16
L = 16
NW = NC * NS

B = 16384
D = 32
CHUNK = 128
BPW = B // NW
NCHUNK = BPW // CHUNK

_mesh = plsc.VectorSubcoreMesh(core_axis_name="c", subcore_axis_name="s")


@functools.partial(
    pl.kernel,
    mesh=_mesh,
    compiler_params=pltpu.CompilerParams(
        needs_layout_passes=False, use_tc_tiling_on_sc=False),
    out_type=jax.ShapeDtypeStruct((B,), jnp.float32),
    scratch_types=[
        pltpu.VMEM((BPW,), jnp.int32),
        pltpu.VMEM((BPW,), jnp.int32),
        pltpu.VMEM((NCHUNK, CHUNK), jnp.int32),
        pltpu.VMEM((NCHUNK, CHUNK), jnp.int32),
        pltpu.VMEM((BPW, D), jnp.float32),
        pltpu.VMEM((BPW, D), jnp.float32),
        pltpu.VMEM((BPW,), jnp.float32),
        pltpu.SemaphoreType.DMA,
    ],
)
def _sc_embed_dot(x0_hbm, x1_hbm, table_hbm, out_hbm,
                  x0_v, x1_v, idx0_v, idx1_v, rows0_v, rows1_v, out_v, sem):
    wid = lax.axis_index("s") * NC + lax.axis_index("c")
    base = wid * BPW

    pltpu.sync_copy(x0_hbm.at[pl.ds(base, BPW)], x0_v)
    pltpu.sync_copy(x1_hbm.at[pl.ds(base, BPW)], x1_v)

    for g in range(BPW // L):
        a = x0_v[pl.ds(g * L, L)]
        b = x1_v[pl.ds(g * L, L)]
        c = g // (CHUNK // L)
        j = g % (CHUNK // L)
        idx0_v[c, pl.ds(j * L, L)] = a
        idx1_v[c, pl.ds(j * L, L)] = a + b

    copies = []
    for c in range(NCHUNK):
        dst0 = rows0_v.at[pl.ds(c * CHUNK, CHUNK)]
        dst1 = rows1_v.at[pl.ds(c * CHUNK, CHUNK)]
        copies.append(pltpu.async_copy(table_hbm.at[idx0_v.at[c]], dst0, sem))
        copies.append(pltpu.async_copy(table_hbm.at[idx1_v.at[c]], dst1, sem))
    for h in copies:
        h.wait()

    lanes = lax.iota(jnp.int32, L)

    def group_body(g, _):
        acc = jnp.zeros((L,), jnp.float32)
        for r in range(L):
            row = g * L + r
            a0 = rows0_v[row, pl.ds(0, L)]
            a1 = rows0_v[row, pl.ds(L, L)]
            b0 = rows1_v[row, pl.ds(0, L)]
            b1 = rows1_v[row, pl.ds(L, L)]
            s = jnp.sum(a0 * b0 + a1 * b1)
            acc = jnp.where(lanes == r, s, acc)
        out_v[pl.ds(g * L, L)] = acc
        return 0
    lax.fori_loop(0, BPW // L, group_body, 0)

    pltpu.sync_copy(out_v, out_hbm.at[pl.ds(base, BPW)])


def kernel(x, table):
    x0 = x[:, 0]
    x1 = x[:, 1]
    return _sc_embed_dot(x0, x1, table)
